# baseline (speedup 1.0000x reference)
import jax
import jax.numpy as jnp
from jax import lax
from jax.experimental import pallas as pl
from jax.experimental.pallas import tpu as pltpu
from jax.experimental.pallas import tpu_sc as plsc

B, S, H = 128, 512, 768
EPS = 1e-12
NC, NS, L = 2, 16, 16
NW = NC * NS
SBLK = S // NW
INV_H = 1.0 / H


def _rsqrt_vec(t):
    i = lax.bitcast_convert_type(t, jnp.int32)
    i = jnp.int32(0x5F3759DF) - lax.shift_right_logical(i, 1)
    y = lax.bitcast_convert_type(i, jnp.float32)
    for _ in range(4):
        y = y * (1.5 - 0.5 * t * y * y)
    return y


def _body(ids_hbm, wemb_hbm, pos_hbm, type_hbm, gamma_hbm, beta_hbm,
          out_hbm, idx_v, rows_v, pe_v, pet_v, ty_v, g_v, b_v, gsem):
    wid = lax.axis_index("s") * NC + lax.axis_index("c")
    sbase = wid * SBLK
    lanes = lax.iota(jnp.int32, L)

    pltpu.sync_copy(pos_hbm.at[pl.ds(sbase, SBLK)], pe_v)
    pltpu.sync_copy(type_hbm.at[pl.ds(0, 1)], ty_v)
    pltpu.sync_copy(gamma_hbm, g_v)
    pltpu.sync_copy(beta_hbm, b_v)

    def pet_col(c, _):
        cidx = jnp.full((L,), c, jnp.int32)
        col = plsc.load_gather(pe_v, [lanes, cidx])
        ty = plsc.load_gather(ty_v, [jnp.zeros((L,), jnp.int32), cidx])
        pet_v[c, pl.ds(0, L)] = col + ty
        return 0
    lax.fori_loop(0, H, pet_col, 0)

    zeros = jnp.zeros((L,), jnp.float32)

    def batch_body(b, _):
        base = b * S + sbase
        pltpu.sync_copy(ids_hbm.at[pl.ds(base, SBLK)], idx_v)
        pltpu.async_copy(wemb_hbm.at[idx_v], rows_v, gsem).wait()

        def pass1(c, carry):
            acc, acc2 = carry
            cidx = jnp.full((L,), c, jnp.int32)
            x = plsc.load_gather(rows_v, [lanes, cidx]) + pet_v[c, pl.ds(0, L)]
            plsc.store_scatter(rows_v, [lanes, cidx], x)
            return acc + x, acc2 + x * x

        acc, acc2 = lax.fori_loop(0, H, pass1, (zeros, zeros))
        mean = acc * INV_H
        var = jnp.maximum(acc2 * INV_H - mean * mean, 0.0)
        rinv = _rsqrt_vec(var + EPS)

        def pass2(c, _):
            cidx = jnp.full((L,), c, jnp.int32)
            x = plsc.load_gather(rows_v, [lanes, cidx])
            g = plsc.load_gather(g_v, [cidx])
            bb = plsc.load_gather(b_v, [cidx])
            y = (x - mean) * rinv * g + bb
            plsc.store_scatter(rows_v, [lanes, cidx], y)
            return 0

        lax.fori_loop(0, H, pass2, 0)
        pltpu.sync_copy(rows_v, out_hbm.at[pl.ds(base, SBLK)])
        return 0

    lax.fori_loop(0, B, batch_body, 0)


@jax.jit
def kernel(input_ids, word_emb, pos_emb, type_emb, ln_gamma, ln_beta):
    ids = input_ids.reshape(B * S).astype(jnp.int32)
    mesh = plsc.VectorSubcoreMesh(core_axis_name="c", subcore_axis_name="s")
    out = pl.kernel(
        _body,
        mesh=mesh,
        compiler_params=pltpu.CompilerParams(
            use_tc_tiling_on_sc=False, needs_layout_passes=False),
        out_type=jax.ShapeDtypeStruct((B * S, H), jnp.float32),
        scratch_types=[
            pltpu.VMEM((SBLK,), jnp.int32),
            pltpu.VMEM((SBLK, H), jnp.float32),
            pltpu.VMEM((SBLK, H), jnp.float32),
            pltpu.VMEM((H, L), jnp.float32),
            pltpu.VMEM((1, H), jnp.float32),
            pltpu.VMEM((H,), jnp.float32),
            pltpu.VMEM((H,), jnp.float32),
            pltpu.SemaphoreType.DMA,
        ],
    )(ids, word_emb, pos_emb, type_emb, ln_gamma, ln_beta)
    return out.reshape(B, S, H)


# speedup vs baseline: 2.3279x; 2.3279x over previous
import jax
import jax.numpy as jnp
from jax import lax
from jax.experimental import pallas as pl
from jax.experimental.pallas import tpu as pltpu
from jax.experimental.pallas import tpu_sc as plsc

B, S, H = 128, 512, 768
EPS = 1e-12
NC, NS, L = 2, 16, 16
NW = NC * NS
SBLK = S // NW
INV_H = 1.0 / H


def _rsqrt_vec(t):
    i = lax.bitcast_convert_type(t, jnp.int32)
    i = jnp.int32(0x5F3759DF) - lax.shift_right_logical(i, 1)
    y = lax.bitcast_convert_type(i, jnp.float32)
    for _ in range(4):
        y = y * (1.5 - 0.5 * t * y * y)
    return y


def _body(ids_hbm, wemb_hbm, pos_hbm, type_hbm, gamma_hbm, beta_hbm,
          out_hbm, idx_v, rows_v, pe_v, pet_v, gt_v, bt_v, ty_v, g_v, b_v,
          gsem):
    wid = lax.axis_index("s") * NC + lax.axis_index("c")
    sbase = wid * SBLK
    lanes = lax.iota(jnp.int32, L)
    zeros_i = jnp.zeros((L,), jnp.int32)

    def diag(c):
        col = lanes + c
        return jnp.where(col >= H, col - H, col)

    pltpu.sync_copy(pos_hbm.at[pl.ds(sbase, SBLK)], pe_v)
    pltpu.sync_copy(type_hbm.at[pl.ds(0, 1)], ty_v)
    pltpu.sync_copy(gamma_hbm, g_v)
    pltpu.sync_copy(beta_hbm, b_v)

    def setup_col(c, _):
        cidx = diag(c)
        pe_col = plsc.load_gather(pe_v, [lanes, cidx])
        ty_col = plsc.load_gather(ty_v, [zeros_i, cidx])
        pet_v[c, pl.ds(0, L)] = pe_col + ty_col
        gt_v[c, pl.ds(0, L)] = plsc.load_gather(g_v, [cidx])
        bt_v[c, pl.ds(0, L)] = plsc.load_gather(b_v, [cidx])
        return 0
    lax.fori_loop(0, H, setup_col, 0)

    zeros = jnp.zeros((L,), jnp.float32)

    def batch_body(b, _):
        base = b * S + sbase
        pltpu.sync_copy(ids_hbm.at[pl.ds(base, SBLK)], idx_v)
        pltpu.async_copy(wemb_hbm.at[idx_v], rows_v, gsem).wait()

        def pass1(c, carry):
            acc, acc2 = carry
            cidx = diag(c)
            x = plsc.load_gather(rows_v, [lanes, cidx]) + pet_v[c, pl.ds(0, L)]
            plsc.store_scatter(rows_v, [lanes, cidx], x)
            return acc + x, acc2 + x * x

        acc, acc2 = lax.fori_loop(0, H, pass1, (zeros, zeros), unroll=4)
        mean = acc * INV_H
        var = jnp.maximum(acc2 * INV_H - mean * mean, 0.0)
        rinv = _rsqrt_vec(var + EPS)

        def pass2(c, _):
            cidx = diag(c)
            x = plsc.load_gather(rows_v, [lanes, cidx])
            y = (x - mean) * rinv * gt_v[c, pl.ds(0, L)] + bt_v[c, pl.ds(0, L)]
            plsc.store_scatter(rows_v, [lanes, cidx], y)
            return 0

        lax.fori_loop(0, H, pass2, 0, unroll=4)
        pltpu.sync_copy(rows_v, out_hbm.at[pl.ds(base, SBLK)])
        return 0

    lax.fori_loop(0, B, batch_body, 0)


@jax.jit
def kernel(input_ids, word_emb, pos_emb, type_emb, ln_gamma, ln_beta):
    ids = input_ids.reshape(B * S).astype(jnp.int32)
    mesh = plsc.VectorSubcoreMesh(core_axis_name="c", subcore_axis_name="s")
    out = pl.kernel(
        _body,
        mesh=mesh,
        compiler_params=pltpu.CompilerParams(
            use_tc_tiling_on_sc=False, needs_layout_passes=False),
        out_type=jax.ShapeDtypeStruct((B * S, H), jnp.float32),
        scratch_types=[
            pltpu.VMEM((SBLK,), jnp.int32),
            pltpu.VMEM((SBLK, H), jnp.float32),
            pltpu.VMEM((SBLK, H), jnp.float32),
            pltpu.VMEM((H, L), jnp.float32),
            pltpu.VMEM((H, L), jnp.float32),
            pltpu.VMEM((H, L), jnp.float32),
            pltpu.VMEM((1, H), jnp.float32),
            pltpu.VMEM((H,), jnp.float32),
            pltpu.VMEM((H,), jnp.float32),
            pltpu.SemaphoreType.DMA,
        ],
    )(ids, word_emb, pos_emb, type_emb, ln_gamma, ln_beta)
    return out.reshape(B, S, H)


# speedup vs baseline: 2.6445x; 1.1360x over previous
import jax
import jax.numpy as jnp
from jax import lax
from jax.experimental import pallas as pl
from jax.experimental.pallas import tpu as pltpu
from jax.experimental.pallas import tpu_sc as plsc

B, S, H = 128, 512, 768
EPS = 1e-12
NC, NS, L = 2, 16, 16
NW = NC * NS
SBLK = S // NW
NCH = H // L
INV_H = 1.0 / H


def _rsqrt_vec(t):
    i = lax.bitcast_convert_type(t, jnp.int32)
    i = jnp.int32(0x5F3759DF) - lax.shift_right_logical(i, 1)
    y = lax.bitcast_convert_type(i, jnp.float32)
    for _ in range(4):
        y = y * (1.5 - 0.5 * t * y * y)
    return y


def _body(ids_hbm, wemb_hbm, pos_hbm, type_hbm, gamma_hbm, beta_hbm,
          out_hbm, idx_v, rows_v, pe_v, ty_v, g_v, b_v, gsem):
    wid = lax.axis_index("s") * NC + lax.axis_index("c")
    sbase = wid * SBLK

    pltpu.sync_copy(pos_hbm.at[pl.ds(sbase, SBLK)], pe_v)
    pltpu.sync_copy(type_hbm.at[pl.ds(0, 1)], ty_v)
    pltpu.sync_copy(gamma_hbm, g_v)
    pltpu.sync_copy(beta_hbm, b_v)

    def fold_type(r, _):
        def fchunk(j, _):
            c = j * L
            pe_v[r, pl.ds(c, L)] = pe_v[r, pl.ds(c, L)] + ty_v[0, pl.ds(c, L)]
            return 0
        return lax.fori_loop(0, NCH, fchunk, 0)
    lax.fori_loop(0, SBLK, fold_type, 0)

    zeros = jnp.zeros((L,), jnp.float32)

    def batch_body(b, _):
        base = b * S + sbase
        pltpu.sync_copy(ids_hbm.at[pl.ds(base, SBLK)], idx_v)
        pltpu.async_copy(wemb_hbm.at[idx_v], rows_v, gsem).wait()

        def row_body(r, _):
            def pass1(j, carry):
                acc, acc2 = carry
                c = j * L
                x = rows_v[r, pl.ds(c, L)] + pe_v[r, pl.ds(c, L)]
                rows_v[r, pl.ds(c, L)] = x
                return acc + x, acc2 + x * x

            acc, acc2 = lax.fori_loop(0, NCH, pass1, (zeros, zeros),
                                      unroll=8)
            s1 = jnp.sum(acc)
            s2 = jnp.sum(acc2)
            mean = s1 * INV_H
            var = jnp.maximum(s2 * INV_H - mean * mean, 0.0)
            rinv = _rsqrt_vec(jnp.full((L,), var + EPS, jnp.float32))

            def pass2(j, _):
                c = j * L
                x = rows_v[r, pl.ds(c, L)]
                y = (x - mean) * rinv * g_v[pl.ds(c, L)] + b_v[pl.ds(c, L)]
                rows_v[r, pl.ds(c, L)] = y
                return 0

            lax.fori_loop(0, NCH, pass2, 0, unroll=8)
            return 0

        lax.fori_loop(0, SBLK, row_body, 0)
        pltpu.sync_copy(rows_v, out_hbm.at[pl.ds(base, SBLK)])
        return 0

    lax.fori_loop(0, B, batch_body, 0)


@jax.jit
def kernel(input_ids, word_emb, pos_emb, type_emb, ln_gamma, ln_beta):
    ids = input_ids.reshape(B * S).astype(jnp.int32)
    mesh = plsc.VectorSubcoreMesh(core_axis_name="c", subcore_axis_name="s")
    out = pl.kernel(
        _body,
        mesh=mesh,
        compiler_params=pltpu.CompilerParams(
            use_tc_tiling_on_sc=False, needs_layout_passes=False),
        out_type=jax.ShapeDtypeStruct((B * S, H), jnp.float32),
        scratch_types=[
            pltpu.VMEM((SBLK,), jnp.int32),
            pltpu.VMEM((SBLK, H), jnp.float32),
            pltpu.VMEM((SBLK, H), jnp.float32),
            pltpu.VMEM((1, H), jnp.float32),
            pltpu.VMEM((H,), jnp.float32),
            pltpu.VMEM((H,), jnp.float32),
            pltpu.SemaphoreType.DMA,
        ],
    )(ids, word_emb, pos_emb, type_emb, ln_gamma, ln_beta)
    return out.reshape(B, S, H)


# speedup vs baseline: 4.3057x; 1.6282x over previous
import jax
import jax.numpy as jnp
from jax import lax
from jax.experimental import pallas as pl
from jax.experimental.pallas import tpu as pltpu
from jax.experimental.pallas import tpu_sc as plsc

B, S, H = 128, 512, 768
EPS = 1e-12
NC, NS, L = 2, 16, 16
NW = NC * NS
SBLK = S // NW
NCH = H // L
INV_H = 1.0 / H


def _rsqrt_vec(t):
    i = lax.bitcast_convert_type(t, jnp.int32)
    i = jnp.int32(0x5F3759DF) - lax.shift_right_logical(i, 1)
    y = lax.bitcast_convert_type(i, jnp.float32)
    for _ in range(4):
        y = y * (1.5 - 0.5 * t * y * y)
    return y


def _body(ids_hbm, wemb_hbm, pos_hbm, type_hbm, gamma_hbm, beta_hbm,
          out_hbm, idx_v, rows_v, pe_v, ty_v, g_v, b_v, gsem):
    wid = lax.axis_index("s") * NC + lax.axis_index("c")
    sbase = wid * SBLK

    pltpu.sync_copy(pos_hbm.at[pl.ds(sbase, SBLK)], pe_v)
    pltpu.sync_copy(type_hbm.at[pl.ds(0, 1)], ty_v)
    pltpu.sync_copy(gamma_hbm, g_v)
    pltpu.sync_copy(beta_hbm, b_v)

    def fold_type(r, _):
        def fchunk(j, _):
            c = j * L
            pe_v[r, pl.ds(c, L)] = pe_v[r, pl.ds(c, L)] + ty_v[0, pl.ds(c, L)]
            return 0
        return lax.fori_loop(0, NCH, fchunk, 0)
    lax.fori_loop(0, SBLK, fold_type, 0)

    zeros = jnp.zeros((L,), jnp.float32)

    def batch_body(b, _):
        base = b * S + sbase
        pltpu.sync_copy(ids_hbm.at[pl.ds(base, SBLK)], idx_v)
        pltpu.async_copy(wemb_hbm.at[idx_v], rows_v, gsem).wait()

        @plsc.parallel_loop(0, SBLK)
        def row_body(r):
            @plsc.parallel_loop(0, H, step=L, unroll=8,
                                carry=(zeros, zeros))
            def moments(c, carry):
                acc, acc2 = carry
                x = rows_v[r, pl.ds(c, L)] + pe_v[r, pl.ds(c, L)]
                rows_v[r, pl.ds(c, L)] = x
                return acc + x, acc2 + x * x

            acc, acc2 = moments
            s1 = jnp.sum(acc)
            s2 = jnp.sum(acc2)
            mean = s1 * INV_H
            var = jnp.maximum(s2 * INV_H - mean * mean, 0.0)
            rinv = _rsqrt_vec(jnp.full((L,), var + EPS, jnp.float32))

            @plsc.parallel_loop(0, H, step=L, unroll=8)
            def pass2(c):
                x = rows_v[r, pl.ds(c, L)]
                y = (x - mean) * rinv * g_v[pl.ds(c, L)] + b_v[pl.ds(c, L)]
                rows_v[r, pl.ds(c, L)] = y
        pltpu.sync_copy(rows_v, out_hbm.at[pl.ds(base, SBLK)])
        return 0

    lax.fori_loop(0, B, batch_body, 0)


@jax.jit
def kernel(input_ids, word_emb, pos_emb, type_emb, ln_gamma, ln_beta):
    ids = input_ids.reshape(B * S).astype(jnp.int32)
    mesh = plsc.VectorSubcoreMesh(core_axis_name="c", subcore_axis_name="s")
    out = pl.kernel(
        _body,
        mesh=mesh,
        compiler_params=pltpu.CompilerParams(
            use_tc_tiling_on_sc=False, needs_layout_passes=False),
        out_type=jax.ShapeDtypeStruct((B * S, H), jnp.float32),
        scratch_types=[
            pltpu.VMEM((SBLK,), jnp.int32),
            pltpu.VMEM((SBLK, H), jnp.float32),
            pltpu.VMEM((SBLK, H), jnp.float32),
            pltpu.VMEM((1, H), jnp.float32),
            pltpu.VMEM((H,), jnp.float32),
            pltpu.VMEM((H,), jnp.float32),
            pltpu.SemaphoreType.DMA,
        ],
    )(ids, word_emb, pos_emb, type_emb, ln_gamma, ln_beta)
    return out.reshape(B, S, H)


# speedup vs baseline: 5.6441x; 1.3108x over previous
import jax
import jax.numpy as jnp
from jax import lax
from jax.experimental import pallas as pl
from jax.experimental.pallas import tpu as pltpu
from jax.experimental.pallas import tpu_sc as plsc

B, S, H = 128, 512, 768
EPS = 1e-12
NC, NS, L = 2, 16, 16
NW = NC * NS
SBLK = S // NW
INV_H = 1.0 / H


def _rsqrt_vec(t):
    i = lax.bitcast_convert_type(t, jnp.int32)
    i = jnp.int32(0x5F3759DF) - lax.shift_right_logical(i, 1)
    y = lax.bitcast_convert_type(i, jnp.float32)
    for _ in range(4):
        y = y * (1.5 - 0.5 * t * y * y)
    return y


def _body(ids_hbm, wemb_hbm, pos_hbm, type_hbm, gamma_hbm, beta_hbm,
          out_hbm, idx_v, rows0_v, rows1_v, y0_v, y1_v, pe_v, ty_v, g_v, b_v,
          gsem0, gsem1, ssem0, ssem1):
    wid = lax.axis_index("s") * NC + lax.axis_index("c")
    sbase = wid * SBLK

    pltpu.sync_copy(ids_hbm.at[:, pl.ds(sbase, SBLK)], idx_v)
    pltpu.sync_copy(pos_hbm.at[pl.ds(sbase, SBLK)], pe_v)
    pltpu.sync_copy(type_hbm.at[pl.ds(0, 1)], ty_v)
    pltpu.sync_copy(gamma_hbm, g_v)
    pltpu.sync_copy(beta_hbm, b_v)

    @plsc.parallel_loop(0, SBLK)
    def fold_type(r):
        @plsc.parallel_loop(0, H, step=L, unroll=4)
        def fchunk(c):
            pe_v[r, pl.ds(c, L)] = pe_v[r, pl.ds(c, L)] + ty_v[0, pl.ds(c, L)]

    zeros = jnp.zeros((L,), jnp.float32)
    bufs = ((rows0_v, y0_v, gsem0, ssem0), (rows1_v, y1_v, gsem1, ssem1))

    def gather_start(b, rows_v, gsem):
        pltpu.make_async_copy(wemb_hbm.at[idx_v.at[b]], rows_v, gsem).start()

    gather_start(0, rows0_v, gsem0)
    gather_start(1, rows1_v, gsem1)

    def step(b, rows_v, y_v, gsem, ssem):
        pltpu.make_async_copy(wemb_hbm.at[idx_v.at[b]], rows_v, gsem).wait()

        @pl.when(b >= 2)
        def _():
            pltpu.make_async_copy(
                y_v, out_hbm.at[pl.ds((b - 2) * S + sbase, SBLK)], ssem
            ).wait()

        @plsc.parallel_loop(0, SBLK)
        def row_body(r):
            @plsc.parallel_loop(0, H, step=L, unroll=8, carry=(zeros, zeros))
            def moments(c, carry):
                acc, acc2 = carry
                x = rows_v[r, pl.ds(c, L)] + pe_v[r, pl.ds(c, L)]
                y_v[r, pl.ds(c, L)] = x
                return acc + x, acc2 + x * x

            acc, acc2 = moments
            s1 = jnp.sum(acc)
            s2 = jnp.sum(acc2)
            mean = s1 * INV_H
            var = jnp.maximum(s2 * INV_H - mean * mean, 0.0)
            rinv = _rsqrt_vec(jnp.full((L,), var + EPS, jnp.float32))

            @plsc.parallel_loop(0, H, step=L, unroll=8)
            def pass2(c):
                x = y_v[r, pl.ds(c, L)]
                y = (x - mean) * rinv * g_v[pl.ds(c, L)] + b_v[pl.ds(c, L)]
                y_v[r, pl.ds(c, L)] = y

        pltpu.make_async_copy(
            y_v, out_hbm.at[pl.ds(b * S + sbase, SBLK)], ssem
        ).start()

        @pl.when(b + 2 < B)
        def _():
            gather_start(b + 2, rows_v, gsem)

    def loop_body(i, _):
        b = i * 2
        step(b, *bufs[0])
        step(b + 1, *bufs[1])
        return 0

    lax.fori_loop(0, B // 2, loop_body, 0)

    pltpu.make_async_copy(
        y0_v, out_hbm.at[pl.ds((B - 2) * S + sbase, SBLK)], ssem0).wait()
    pltpu.make_async_copy(
        y1_v, out_hbm.at[pl.ds((B - 1) * S + sbase, SBLK)], ssem1).wait()


@jax.jit
def kernel(input_ids, word_emb, pos_emb, type_emb, ln_gamma, ln_beta):
    ids = input_ids.astype(jnp.int32)
    mesh = plsc.VectorSubcoreMesh(core_axis_name="c", subcore_axis_name="s")
    out = pl.kernel(
        _body,
        mesh=mesh,
        compiler_params=pltpu.CompilerParams(
            use_tc_tiling_on_sc=False, needs_layout_passes=False),
        out_type=jax.ShapeDtypeStruct((B * S, H), jnp.float32),
        scratch_types=[
            pltpu.VMEM((B, SBLK), jnp.int32),
            pltpu.VMEM((SBLK, H), jnp.float32),
            pltpu.VMEM((SBLK, H), jnp.float32),
            pltpu.VMEM((SBLK, H), jnp.float32),
            pltpu.VMEM((SBLK, H), jnp.float32),
            pltpu.VMEM((SBLK, H), jnp.float32),
            pltpu.VMEM((1, H), jnp.float32),
            pltpu.VMEM((H,), jnp.float32),
            pltpu.VMEM((H,), jnp.float32),
            pltpu.SemaphoreType.DMA,
            pltpu.SemaphoreType.DMA,
            pltpu.SemaphoreType.DMA,
            pltpu.SemaphoreType.DMA,
        ],
    )(ids, word_emb, pos_emb, type_emb, ln_gamma, ln_beta)
    return out.reshape(B, S, H)
